# baseline (speedup 1.0000x reference)
import functools

import jax
import jax.numpy as jnp
from jax import lax
from jax.experimental import pallas as pl
from jax.experimental.pallas import tpu as pltpu
from jax.experimental.pallas import tpu_sc as plsc

BLANK = 0
DEPTH = 1000
T_DIM = 50
B_DIM = 1024
CHUNK = 40
N_CHUNKS = DEPTH // CHUNK
N_UNITS = T_DIM * N_CHUNKS
N_VREGS = B_DIM // 16
MAX_PAIRS = 20


def _sc_onehot(idx_hbm, out_hbm, bufs, ids, sems):
    info = plsc.get_sparse_core_info()
    nc = info.num_cores
    ns = info.num_subcores
    nw = nc * ns
    wid = lax.axis_index("s") * nc + lax.axis_index("c")

    base_units = N_UNITS // nw
    n_extra = N_UNITS - base_units * nw
    lo = base_units * wid + jnp.minimum(wid, n_extra)
    cnt = base_units + jnp.where(wid < n_extra, 1, 0)

    lane = lax.broadcasted_iota(jnp.int32, (16,), 0)
    ones16 = jnp.full((16,), 1.0, dtype=jnp.float32)
    zeros16 = jnp.zeros((16,), dtype=jnp.float32)

    for b in range(2):
        def _zero_row(r, _):
            for c in range(N_VREGS):
                bufs[b, r, pl.ds(c * 16, 16)] = zeros16
            return 0
        lax.fori_loop(0, CHUNK, _zero_row, 0)

    def _scatter(b, c0, vals):
        def _step(v, _):
            vec = ids[b, pl.ds(v * 16, 16)]
            m = (vec > 0) & (vec >= c0) & (vec < c0 + CHUNK)
            plsc.store_scatter(
                bufs.at[b], [vec - c0, v * 16 + lane], vals, mask=m)
            return 0
        lax.fori_loop(0, N_VREGS, _step, 0)

    def _pair(p, _):
        for b in range(2):
            kk = 2 * p + b
            u = lo + kk

            @pl.when(kk < cnt)
            def _do_unit():
                t = u // N_CHUNKS
                c0 = (u % N_CHUNKS) * CHUNK

                @pl.when(p > 0)
                def _recycle():
                    u_prev = u - 2
                    t_prev = u_prev // N_CHUNKS
                    c0_prev = (u_prev % N_CHUNKS) * CHUNK
                    pltpu.make_async_copy(
                        bufs.at[b],
                        out_hbm.at[t_prev, pl.ds(c0_prev, CHUNK)],
                        sems.at[b],
                    ).wait()
                    _scatter(b, c0_prev, zeros16)

                pltpu.sync_copy(idx_hbm.at[t], ids.at[b])
                _scatter(b, c0, ones16)
                pltpu.make_async_copy(
                    bufs.at[b],
                    out_hbm.at[t, pl.ds(c0, CHUNK)],
                    sems.at[b],
                ).start()
        return 0

    lax.fori_loop(0, MAX_PAIRS, _pair, 0)

    for b in range(2):
        kl = cnt - 1 - ((cnt - 1 - b) % 2)
        u = lo + kl
        t = u // N_CHUNKS
        c0 = (u % N_CHUNKS) * CHUNK
        pltpu.make_async_copy(
            bufs.at[b],
            out_hbm.at[t, pl.ds(c0, CHUNK)],
            sems.at[b],
        ).wait()


def kernel(outputs, outputs_length):
    idx_t = outputs.astype(jnp.int32).T
    mesh = plsc.VectorSubcoreMesh(core_axis_name="c", subcore_axis_name="s")
    sc_call = functools.partial(
        pl.kernel,
        mesh=mesh,
        compiler_params=pltpu.CompilerParams(needs_layout_passes=False),
        out_type=jax.ShapeDtypeStruct((T_DIM, DEPTH, B_DIM), jnp.float32),
        scratch_types=[
            pltpu.VMEM((2, CHUNK, B_DIM), jnp.float32),
            pltpu.VMEM((2, B_DIM), jnp.int32),
            pltpu.SemaphoreType.DMA((2,)),
        ],
    )(_sc_onehot)
    one_hot_t = sc_call(idx_t)
    return (jnp.transpose(one_hot_t, (2, 0, 1)), outputs_length)
